# baseline (speedup 1.0000x reference)
import functools

import jax
import jax.numpy as jnp
from jax import lax
from jax.experimental import pallas as pl
from jax.experimental.pallas import tpu as pltpu
from jax.experimental.pallas import tpu_sc as plsc

_BATCH = 16384
_EMBED = 32
_NC = 2
_NS = 16
_NW = _NC * _NS
_BW = _BATCH // _NW
_CHUNK = 128
_NCHUNK = _BW // _CHUNK


def _sc_body(uid_hbm, pid_hbm, uemb_hbm, ubias_hbm, pemb_hbm, pbias_hbm,
             out_hbm, idx_u, idx_p, urows, prows, ub_v, pb_v, out_v, sem):
    wid = lax.axis_index("s") * _NC + lax.axis_index("c")

    pltpu.sync_copy(uid_hbm.at[pl.ds(wid * _NCHUNK, _NCHUNK)], idx_u)
    pltpu.sync_copy(pid_hbm.at[pl.ds(wid * _NCHUNK, _NCHUNK)], idx_p)

    copies = []
    for k in range(_NCHUNK):
        sl = pl.ds(k * _CHUNK, _CHUNK)
        copies.append(pltpu.async_copy(uemb_hbm.at[idx_u.at[k]], urows.at[sl], sem))
        copies.append(pltpu.async_copy(pemb_hbm.at[idx_p.at[k]], prows.at[sl], sem))
        copies.append(pltpu.async_copy(ubias_hbm.at[idx_u.at[k]], ub_v.at[sl], sem))
        copies.append(pltpu.async_copy(pbias_hbm.at[idx_p.at[k]], pb_v.at[sl], sem))
    for c in copies:
        c.wait()

    iota = lax.iota(jnp.int32, 16)
    ecols = [jnp.full((16,), e, jnp.int32) for e in range(_EMBED)]

    def blk_body(blk, carry):
        r0 = blk * 16
        ridx = iota + r0
        acc = ub_v[pl.ds(r0, 16)] + pb_v[pl.ds(r0, 16)]
        for e in range(_EMBED):
            uu = plsc.load_gather(urows, [ridx, ecols[e]])
            pp = plsc.load_gather(prows, [ridx, ecols[e]])
            acc = acc + uu * pp
        out_v[pl.ds(r0, 16)] = acc
        return carry

    lax.fori_loop(0, _BW // 16, blk_body, 0)
    pltpu.sync_copy(out_v, out_hbm.at[pl.ds(wid * _BW, _BW)])


_sc_call = functools.partial(
    pl.kernel,
    out_type=jax.ShapeDtypeStruct((_BATCH,), jnp.float32),
    mesh=plsc.VectorSubcoreMesh(core_axis_name="c", subcore_axis_name="s"),
    compiler_params=pltpu.CompilerParams(
        needs_layout_passes=False, use_tc_tiling_on_sc=False),
    scratch_types=[
        pltpu.VMEM((_NCHUNK, _CHUNK), jnp.int32),
        pltpu.VMEM((_NCHUNK, _CHUNK), jnp.int32),
        pltpu.VMEM((_BW, _EMBED), jnp.float32),
        pltpu.VMEM((_BW, _EMBED), jnp.float32),
        pltpu.VMEM((_BW,), jnp.float32),
        pltpu.VMEM((_BW,), jnp.float32),
        pltpu.VMEM((_BW,), jnp.float32),
        pltpu.SemaphoreType.DMA,
    ],
)(_sc_body)


@jax.jit
def kernel(inputs, user_emb, user_bias, place_emb, place_bias):
    uid = inputs[:, 0].astype(jnp.int32).reshape(_NW * _NCHUNK, _CHUNK)
    pid = inputs[:, 1].astype(jnp.int32).reshape(_NW * _NCHUNK, _CHUNK)
    ubias = user_bias.reshape(-1)
    pbias = place_bias.reshape(-1)
    return _sc_call(uid, pid, user_emb, ubias, place_emb, pbias)
